# baseline (speedup 1.0000x reference)
import functools

import jax
import jax.numpy as jnp
from jax.experimental import pallas as pl
from jax.experimental.pallas import tpu as pltpu

DIM = 1024
HID = 2048
NE = 8
NTOK = 2048


def _gate_body(x_ref, wg_ref, bg_ref, loss_ref, cw_ref):
    x = x_ref[...]
    logits = jnp.dot(x, wg_ref[...], preferred_element_type=jnp.float32)
    logits = logits + bg_ref[...]
    m = jnp.max(logits, axis=-1, keepdims=True)
    ex = jnp.exp(logits - m)
    gw = ex / jnp.sum(ex, axis=-1, keepdims=True)

    lane = jax.lax.broadcasted_iota(jnp.int32, (NTOK, NE), 1)
    m1 = jnp.max(gw, axis=-1, keepdims=True)
    i1 = jnp.min(jnp.where(gw == m1, lane, NE), axis=-1, keepdims=True)
    sel1 = lane == i1
    gw2 = jnp.where(sel1, -jnp.inf, gw)
    m2 = jnp.max(gw2, axis=-1, keepdims=True)
    i2 = jnp.min(jnp.where(gw2 == m2, lane, NE), axis=-1, keepdims=True)
    sel2 = lane == i2

    wsum = m1 + m2
    w1 = m1 / wsum
    w2 = m2 / wsum
    cw_ref[...] = jnp.where(sel1, w1, 0.0) + jnp.where(sel2, w2, 0.0)

    usage = jnp.sum(jnp.where(sel1 | sel2, 1.0, 0.0), axis=0, keepdims=True)
    prob = jnp.sum(gw, axis=0, keepdims=True) / NTOK
    loss_ref[...] = jnp.sum(prob * usage / NTOK).reshape(1, 1)


def _gate(xf, Wg, bg):
    return pl.pallas_call(
        _gate_body,
        out_shape=(
            jax.ShapeDtypeStruct((1, 1), jnp.float32),
            jax.ShapeDtypeStruct((NTOK, NE), jnp.float32),
        ),
    )(xf, Wg, bg.reshape(1, NE))


def _dense_body(x_ref, w1_ref, b1_ref, w2_ref, b2_ref, cw_ref, out_ref):
    t = pl.program_id(0)
    e = pl.program_id(1)
    blk = x_ref.shape[0]

    cwb = cw_ref[...]
    lane = jax.lax.broadcasted_iota(jnp.int32, (blk, NE), 1)
    wcol = jnp.sum(jnp.where(lane == e, cwb, 0.0), axis=-1, keepdims=True)

    h = jnp.dot(x_ref[...], w1_ref[0], preferred_element_type=jnp.float32)
    h = jnp.maximum(h + b1_ref[0], 0.0)
    y = jnp.dot(h, w2_ref[0], preferred_element_type=jnp.float32)
    y = (y + b2_ref[0]) * wcol

    @pl.when(e == 0)
    def _():
        out_ref[...] = jnp.zeros_like(out_ref)

    out_ref[...] += y


def _dense_moe(xf, W1, b1, W2, b2, cw):
    blk = 256
    T = NTOK // blk
    return pl.pallas_call(
        _dense_body,
        grid=(T, NE),
        in_specs=[
            pl.BlockSpec((blk, DIM), lambda t, e: (t, 0)),
            pl.BlockSpec((1, DIM, HID), lambda t, e: (e, 0, 0)),
            pl.BlockSpec((1, 1, HID), lambda t, e: (e, 0, 0)),
            pl.BlockSpec((1, HID, DIM), lambda t, e: (e, 0, 0)),
            pl.BlockSpec((1, 1, DIM), lambda t, e: (e, 0, 0)),
            pl.BlockSpec((blk, NE), lambda t, e: (t, 0)),
        ],
        out_specs=pl.BlockSpec((blk, DIM), lambda t, e: (t, 0)),
        out_shape=jax.ShapeDtypeStruct((NTOK, DIM), jnp.float32),
    )(xf, W1, b1.reshape(NE, 1, HID), W2, b2.reshape(NE, 1, DIM), cw)


def kernel(x, W1, b1, W2, b2, Wg, bg):
    orig_shape = x.shape
    xf = x.reshape(-1, orig_shape[-1])
    loss, cw = _gate(xf, Wg, bg)
    out = _dense_moe(xf, W1, b1, W2, b2, cw)
    return out.reshape(orig_shape), loss.reshape(())


# speedup vs baseline: 1.8282x; 1.8282x over previous
import functools

import jax
import jax.numpy as jnp
from jax import lax
from jax.experimental import pallas as pl
from jax.experimental.pallas import tpu as pltpu
from jax.experimental.pallas import tpu_sc as plsc

DIM = 1024
HID = 2048
NE = 8
NTOK = 2048
NA = 2 * NTOK
BLK = 256
NT = NA // BLK
NS = 32

NSC_CORES = 2
NSC_SUB = 16
NW = NSC_CORES * NSC_SUB


def _tr8(row):
    eye = (jax.lax.broadcasted_iota(jnp.int32, (NE, NE), 0)
           == jax.lax.broadcasted_iota(jnp.int32, (NE, NE), 1)).astype(jnp.float32)
    return jnp.sum(jnp.broadcast_to(row, (NE, NE)) * eye, axis=1, keepdims=True)


def _gate_body(x_ref, wg_ref, bg_ref, loss_ref, pos_ref, w01_ref, meta_ref):
    x = x_ref[...]
    logits = jnp.dot(x, wg_ref[...], preferred_element_type=jnp.float32)
    logits = logits + bg_ref[...]
    m = jnp.max(logits, axis=-1, keepdims=True)
    ex = jnp.exp(logits - m)
    gw = ex / jnp.sum(ex, axis=-1, keepdims=True)

    lane = jax.lax.broadcasted_iota(jnp.int32, (NTOK, NE), 1)
    m1 = jnp.max(gw, axis=-1, keepdims=True)
    i1 = jnp.min(jnp.where(gw == m1, lane, NE), axis=-1, keepdims=True)
    sel1 = lane == i1
    gw2 = jnp.where(sel1, -jnp.inf, gw)
    m2 = jnp.max(gw2, axis=-1, keepdims=True)
    i2 = jnp.min(jnp.where(gw2 == m2, lane, NE), axis=-1, keepdims=True)
    sel2 = lane == i2

    wsum = m1 + m2
    w01_ref[...] = jnp.broadcast_to(
        jnp.concatenate([m1 / wsum, m2 / wsum], axis=0), (NA, 16))

    usage = jnp.sum(jnp.where(sel1 | sel2, 1.0, 0.0), axis=0, keepdims=True)
    prob = jnp.sum(gw, axis=0, keepdims=True) / NTOK
    loss_ref[...] = jnp.sum(prob * usage / NTOK).reshape(1, 1)

    oh = jnp.concatenate([sel1, sel2], axis=0).astype(jnp.float32)
    c = oh
    sh = 1
    while sh < NA:
        c = c + jnp.concatenate([jnp.zeros((sh, NE), jnp.float32), c[:-sh]], axis=0)
        sh *= 2
    rank = jnp.sum(oh * (c - oh), axis=1, keepdims=True)
    counts = jnp.sum(oh, axis=0, keepdims=True)

    r8 = jax.lax.broadcasted_iota(jnp.int32, (NE, NE), 0)
    c8 = jax.lax.broadcasted_iota(jnp.int32, (NE, NE), 1)
    strict_lt = (r8 < c8).astype(jnp.float32)

    def _excl_prefix_row(row):
        col = _tr8(row)
        return jnp.sum(jnp.broadcast_to(col, (NE, NE)) * strict_lt,
                       axis=0, keepdims=True)

    offs = _excl_prefix_row(counts)

    posf = rank + jnp.sum(oh * offs, axis=1, keepdims=True)
    pos_ref[...] = posf.astype(jnp.int32)

    t_start = jnp.floor(offs / BLK)
    endf = offs + counts
    t_end = jnp.floor((endf + (BLK - 1)) / BLK)
    nt = jnp.where(counts > 0, t_end - t_start, 0.0)
    so = _excl_prefix_row(nt)
    ce = so + nt
    total = jnp.sum(nt)

    ce_col = _tr8(ce)
    so_col = _tr8(so)
    ts_col = _tr8(t_start)
    off_col = _tr8(offs)
    end_col = _tr8(endf)

    siota = jax.lax.broadcasted_iota(jnp.int32, (1, NS), 1).astype(jnp.float32)
    pad = siota >= total
    s_c = jnp.minimum(siota, jnp.maximum(total - 1.0, 0.0))
    g = jnp.sum((s_c >= ce_col).astype(jnp.float32), axis=0, keepdims=True)

    e_col = jax.lax.broadcasted_iota(jnp.int32, (NE, 1), 0).astype(jnp.float32)
    gsel = (e_col == g).astype(jnp.float32)
    so_g = jnp.sum(gsel * so_col, axis=0, keepdims=True)
    ts_g = jnp.sum(gsel * ts_col, axis=0, keepdims=True)
    off_g = jnp.sum(gsel * off_col, axis=0, keepdims=True)
    end_g = jnp.sum(gsel * end_col, axis=0, keepdims=True)

    tile = s_c - so_g + ts_g
    lo = jnp.maximum(off_g, tile * BLK)
    hi = jnp.minimum(end_g, (tile + 1.0) * BLK)
    hi = jnp.where(pad, lo, hi)

    meta_ref[...] = jnp.concatenate([tile, g, lo, hi], axis=0).astype(jnp.int32)


def _gate(xf, Wg, bg):
    return pl.pallas_call(
        _gate_body,
        out_shape=(
            jax.ShapeDtypeStruct((1, 1), jnp.float32),
            jax.ShapeDtypeStruct((NA, 1), jnp.int32),
            jax.ShapeDtypeStruct((NA, 16), jnp.float32),
            jax.ShapeDtypeStruct((4, NS), jnp.int32),
        ),
    )(xf, Wg, bg.reshape(1, NE))


@functools.cache
def _sc_kernels():
    mesh = plsc.VectorSubcoreMesh(core_axis_name="c", subcore_axis_name="s")

    @functools.partial(
        pl.kernel,
        out_type=jax.ShapeDtypeStruct((NA, DIM), jnp.float32),
        mesh=mesh,
        scratch_types=[
            pltpu.VMEM((4, 32), jnp.int32),
            pltpu.VMEM((32, DIM), jnp.float32),
            pltpu.SemaphoreType.DMA,
        ],
    )
    def _dispatch(x_hbm, pos_hbm, xs_hbm, idx_v, buf, sem):
        wid = lax.axis_index("s") * NSC_CORES + lax.axis_index("c")
        pltpu.sync_copy(pos_hbm.at[pl.ds(wid * 4, 4)], idx_v)
        for c in range(4):
            j0 = wid * 128 + c * 32
            tok = lax.rem(j0, NTOK)
            pltpu.sync_copy(x_hbm.at[pl.ds(tok, 32)], buf)
            pltpu.async_copy(buf, xs_hbm.at[idx_v.at[c]], sem).wait()

    @functools.partial(
        pl.kernel,
        out_type=jax.ShapeDtypeStruct((NTOK, DIM), jnp.float32),
        mesh=mesh,
        scratch_types=[
            pltpu.VMEM((2, 32), jnp.int32),
            pltpu.VMEM((32, 16), jnp.float32),
            pltpu.VMEM((32, 16), jnp.float32),
            pltpu.VMEM((32, DIM), jnp.float32),
            pltpu.VMEM((32, DIM), jnp.float32),
            pltpu.SemaphoreType.DMA,
        ],
    )
    def _combine(ys_hbm, pos_hbm, w_hbm, out_hbm, idx_v, wv0, wv1, buf0, buf1,
                 sem):
        wid = lax.axis_index("s") * NSC_CORES + lax.axis_index("c")
        for c in range(2):
            n0 = wid * 64 + c * 32
            r0 = 2 * wid + c
            r1 = 64 + r0
            pltpu.sync_copy(pos_hbm.at[pl.ds(r0, 1)], idx_v.at[pl.ds(0, 1)])
            pltpu.sync_copy(pos_hbm.at[pl.ds(r1, 1)], idx_v.at[pl.ds(1, 1)])
            pltpu.sync_copy(w_hbm.at[pl.ds(n0, 32)], wv0)
            pltpu.sync_copy(w_hbm.at[pl.ds(NTOK + n0, 32)], wv1)
            pltpu.async_copy(ys_hbm.at[idx_v.at[0]], buf0, sem).wait()
            pltpu.async_copy(ys_hbm.at[idx_v.at[1]], buf1, sem).wait()

            def row_body(r, carry):
                s0 = wv0[r, :]
                s1 = wv1[r, :]

                def col_body(t, carry2):
                    a = buf0[r, pl.ds(t * 16, 16)]
                    b = buf1[r, pl.ds(t * 16, 16)]
                    buf0[r, pl.ds(t * 16, 16)] = s0 * a + s1 * b
                    return carry2

                return lax.fori_loop(0, DIM // 16, col_body, carry)

            lax.fori_loop(0, 32, row_body, 0)
            pltpu.sync_copy(buf0, out_hbm.at[pl.ds(n0, 32)])

    return _dispatch, _combine


def _gmm_body(tiles, groups, los, his, xs_ref, w1_ref, b1_ref, w2_ref, b2_ref,
              ys_ref):
    s = pl.program_id(0)
    tile = tiles[s]
    prev = tiles[jnp.maximum(s - 1, 0)]
    first = jnp.logical_or(s == 0, tile != prev)
    lo = los[s] - tile * BLK
    hi = his[s] - tile * BLK

    @pl.when(first)
    def _():
        ys_ref[...] = jnp.zeros_like(ys_ref)

    @pl.when(lo < hi)
    def _():
        h = jnp.dot(xs_ref[...], w1_ref[0], preferred_element_type=jnp.float32)
        h = jnp.maximum(h + b1_ref[0], 0.0)
        y = jnp.dot(h, w2_ref[0], preferred_element_type=jnp.float32)
        y = y + b2_ref[0]
        rows = jax.lax.broadcasted_iota(jnp.int32, (BLK, 1), 0)
        mask = (rows >= lo) & (rows < hi)
        ys_ref[...] += jnp.where(mask, y, 0.0)


def _gmm(xs, W1, b1, W2, b2, tiles, groups, los, his):
    grid_spec = pltpu.PrefetchScalarGridSpec(
        num_scalar_prefetch=4,
        grid=(NS,),
        in_specs=[
            pl.BlockSpec((BLK, DIM), lambda s, t, g, l, h: (t[s], 0)),
            pl.BlockSpec((1, DIM, HID), lambda s, t, g, l, h: (g[s], 0, 0)),
            pl.BlockSpec((1, 1, HID), lambda s, t, g, l, h: (g[s], 0, 0)),
            pl.BlockSpec((1, HID, DIM), lambda s, t, g, l, h: (g[s], 0, 0)),
            pl.BlockSpec((1, 1, DIM), lambda s, t, g, l, h: (g[s], 0, 0)),
        ],
        out_specs=pl.BlockSpec((BLK, DIM), lambda s, t, g, l, h: (t[s], 0)),
    )
    return pl.pallas_call(
        _gmm_body,
        grid_spec=grid_spec,
        out_shape=jax.ShapeDtypeStruct((NA, DIM), jnp.float32),
        compiler_params=pltpu.CompilerParams(vmem_limit_bytes=100 * 2**20),
    )(tiles, groups, los, his, xs, W1, b1.reshape(NE, 1, HID), W2,
      b2.reshape(NE, 1, DIM))


def kernel(x, W1, b1, W2, b2, Wg, bg):
    orig_shape = x.shape
    xf = x.reshape(-1, orig_shape[-1])
    dispatch, combine = _sc_kernels()
    loss, position, w01, meta = _gate(xf, Wg, bg)
    pos2d = position.reshape(NA // 32, 32)
    xs = dispatch(xf, pos2d)
    ys = _gmm(xs, W1, b1, W2, b2, meta[0], meta[1], meta[2], meta[3])
    out = combine(ys, pos2d, w01)
    return out.reshape(orig_shape), loss.reshape(())


# speedup vs baseline: 1.9528x; 1.0682x over previous
import functools

import jax
import jax.numpy as jnp
from jax import lax
from jax.experimental import pallas as pl
from jax.experimental.pallas import tpu as pltpu
from jax.experimental.pallas import tpu_sc as plsc

DIM = 1024
HID = 2048
NE = 8
NTOK = 2048
NA = 2 * NTOK
BLK = 256
NT = NA // BLK
NS = 32

NSC_CORES = 2
NSC_SUB = 16
NW = NSC_CORES * NSC_SUB


def _tr8(row):
    eye = (jax.lax.broadcasted_iota(jnp.int32, (NE, NE), 0)
           == jax.lax.broadcasted_iota(jnp.int32, (NE, NE), 1)).astype(jnp.float32)
    return jnp.sum(jnp.broadcast_to(row, (NE, NE)) * eye, axis=1, keepdims=True)


def _gate_body(x_ref, wg_ref, bg_ref, loss_ref, pos_ref, w01_ref, meta_ref):
    x = x_ref[...]
    logits = jnp.dot(x, wg_ref[...], preferred_element_type=jnp.float32)
    logits = logits + bg_ref[...]
    m = jnp.max(logits, axis=-1, keepdims=True)
    ex = jnp.exp(logits - m)
    gw = ex / jnp.sum(ex, axis=-1, keepdims=True)

    lane = jax.lax.broadcasted_iota(jnp.int32, (NTOK, NE), 1)
    m1 = jnp.max(gw, axis=-1, keepdims=True)
    i1 = jnp.min(jnp.where(gw == m1, lane, NE), axis=-1, keepdims=True)
    sel1 = lane == i1
    gw2 = jnp.where(sel1, -jnp.inf, gw)
    m2 = jnp.max(gw2, axis=-1, keepdims=True)
    i2 = jnp.min(jnp.where(gw2 == m2, lane, NE), axis=-1, keepdims=True)
    sel2 = lane == i2

    wsum = m1 + m2
    w01_ref[...] = jnp.broadcast_to(
        jnp.concatenate([m1 / wsum, m2 / wsum], axis=0), (NA, 128))

    usage = jnp.sum(jnp.where(sel1 | sel2, 1.0, 0.0), axis=0, keepdims=True)
    prob = jnp.sum(gw, axis=0, keepdims=True) / NTOK
    loss_ref[...] = jnp.sum(prob * usage / NTOK).reshape(1, 1)

    oh = jnp.concatenate([sel1, sel2], axis=0).astype(jnp.float32)
    CH = 512
    rc = jax.lax.broadcasted_iota(jnp.int32, (CH, CH), 0)
    cc = jax.lax.broadcasted_iota(jnp.int32, (CH, CH), 1)
    ltc = (cc < rc).astype(jnp.float32)
    carry = jnp.zeros((1, NE), jnp.float32)
    rank_chunks = []
    for k in range(NA // CH):
        ohk = oh[k * CH:(k + 1) * CH]
        rk = jnp.dot(ltc, ohk, preferred_element_type=jnp.float32) + carry
        rank_chunks.append(jnp.sum(ohk * rk, axis=1, keepdims=True))
        carry = carry + jnp.sum(ohk, axis=0, keepdims=True)
    rank = jnp.concatenate(rank_chunks, axis=0)
    counts = carry

    r8 = jax.lax.broadcasted_iota(jnp.int32, (NE, NE), 0)
    c8 = jax.lax.broadcasted_iota(jnp.int32, (NE, NE), 1)
    strict_lt = (r8 < c8).astype(jnp.float32)

    def _excl_prefix_row(row):
        col = _tr8(row)
        return jnp.sum(jnp.broadcast_to(col, (NE, NE)) * strict_lt,
                       axis=0, keepdims=True)

    offs = _excl_prefix_row(counts)

    posf = rank + jnp.sum(oh * offs, axis=1, keepdims=True)
    pos_ref[...] = posf.astype(jnp.int32)

    t_start = jnp.floor(offs / BLK)
    endf = offs + counts
    t_end = jnp.floor((endf + (BLK - 1)) / BLK)
    nt = jnp.where(counts > 0, t_end - t_start, 0.0)
    so = _excl_prefix_row(nt)
    ce = so + nt
    total = jnp.sum(nt)

    ce_col = _tr8(ce)
    so_col = _tr8(so)
    ts_col = _tr8(t_start)
    off_col = _tr8(offs)
    end_col = _tr8(endf)

    siota = jax.lax.broadcasted_iota(jnp.int32, (1, NS), 1).astype(jnp.float32)
    pad = siota >= total
    s_c = jnp.minimum(siota, jnp.maximum(total - 1.0, 0.0))
    g = jnp.sum((s_c >= ce_col).astype(jnp.float32), axis=0, keepdims=True)

    e_col = jax.lax.broadcasted_iota(jnp.int32, (NE, 1), 0).astype(jnp.float32)
    gsel = (e_col == g).astype(jnp.float32)
    so_g = jnp.sum(gsel * so_col, axis=0, keepdims=True)
    ts_g = jnp.sum(gsel * ts_col, axis=0, keepdims=True)
    off_g = jnp.sum(gsel * off_col, axis=0, keepdims=True)
    end_g = jnp.sum(gsel * end_col, axis=0, keepdims=True)

    tile = s_c - so_g + ts_g
    lo = jnp.maximum(off_g, tile * BLK)
    hi = jnp.minimum(end_g, (tile + 1.0) * BLK)
    hi = jnp.where(pad, lo, hi)

    meta_ref[...] = jnp.concatenate([tile, g, lo, hi], axis=0).astype(jnp.int32)


def _gate(xf, Wg, bg):
    return pl.pallas_call(
        _gate_body,
        out_shape=(
            jax.ShapeDtypeStruct((1, 1), jnp.float32),
            jax.ShapeDtypeStruct((NA, 1), jnp.int32),
            jax.ShapeDtypeStruct((NA, 128), jnp.float32),
            jax.ShapeDtypeStruct((4, NS), jnp.int32),
        ),
    )(xf, Wg, bg.reshape(1, NE))


@functools.cache
def _sc_kernels():
    mesh = plsc.VectorSubcoreMesh(core_axis_name="c", subcore_axis_name="s")

    @functools.partial(
        pl.kernel,
        out_type=(
            jax.ShapeDtypeStruct((NA, DIM), jnp.float32),
            jax.ShapeDtypeStruct((NA, 128), jnp.float32),
        ),
        mesh=mesh,
        scratch_types=[
            pltpu.VMEM((4, 32), jnp.int32),
            pltpu.VMEM((32, DIM), jnp.float32),
            pltpu.VMEM((32, 128), jnp.float32),
            pltpu.SemaphoreType.DMA,
        ],
    )
    def _dispatch(x_hbm, w_hbm, pos_hbm, xs_hbm, ws_hbm, idx_v, buf, wbuf,
                  sem):
        wid = lax.axis_index("s") * NSC_CORES + lax.axis_index("c")
        pltpu.sync_copy(pos_hbm.at[pl.ds(wid * 4, 4)], idx_v)
        for c in range(4):
            j0 = wid * 128 + c * 32
            tok = lax.rem(j0, NTOK)
            pltpu.sync_copy(x_hbm.at[pl.ds(tok, 32)], buf)
            pltpu.sync_copy(w_hbm.at[pl.ds(j0, 32)], wbuf)
            h0 = pltpu.async_copy(buf, xs_hbm.at[idx_v.at[c]], sem)
            h1 = pltpu.async_copy(wbuf, ws_hbm.at[idx_v.at[c]], sem)
            h0.wait()
            h1.wait()

    @functools.partial(
        pl.kernel,
        out_type=jax.ShapeDtypeStruct((NTOK, DIM), jnp.float32),
        mesh=mesh,
        scratch_types=[
            pltpu.VMEM((2, 32), jnp.int32),
            pltpu.VMEM((32, DIM), jnp.float32),
            pltpu.VMEM((32, DIM), jnp.float32),
            pltpu.SemaphoreType.DMA,
        ],
    )
    def _combine(ys_hbm, pos_hbm, out_hbm, idx_v, buf0, buf1, sem):
        wid = lax.axis_index("s") * NSC_CORES + lax.axis_index("c")
        for c in range(2):
            n0 = wid * 64 + c * 32
            r0 = 2 * wid + c
            r1 = 64 + r0
            pltpu.sync_copy(pos_hbm.at[pl.ds(r0, 1)], idx_v.at[pl.ds(0, 1)])
            pltpu.sync_copy(pos_hbm.at[pl.ds(r1, 1)], idx_v.at[pl.ds(1, 1)])
            h0 = pltpu.async_copy(ys_hbm.at[idx_v.at[0]], buf0, sem)
            h1 = pltpu.async_copy(ys_hbm.at[idx_v.at[1]], buf1, sem)
            h0.wait()
            h1.wait()

            def row_body(r, carry):
                for t in range(DIM // 16):
                    a = buf0[r, pl.ds(t * 16, 16)]
                    b = buf1[r, pl.ds(t * 16, 16)]
                    buf0[r, pl.ds(t * 16, 16)] = a + b
                return carry

            lax.fori_loop(0, 32, row_body, 0)
            pltpu.sync_copy(buf0, out_hbm.at[pl.ds(n0, 32)])

    return _dispatch, _combine


def _gmm_body(tiles, groups, los, his, xs_ref, ws_ref, w1_ref, b1_ref, w2_ref,
              b2_ref, ys_ref):
    s = pl.program_id(0)
    tile = tiles[s]
    prev = tiles[jnp.maximum(s - 1, 0)]
    first = jnp.logical_or(s == 0, tile != prev)
    lo = los[s] - tile * BLK
    hi = his[s] - tile * BLK

    @pl.when(first)
    def _():
        ys_ref[...] = jnp.zeros_like(ys_ref)

    @pl.when(lo < hi)
    def _():
        h = jnp.dot(xs_ref[...], w1_ref[0], preferred_element_type=jnp.float32)
        h = jnp.maximum(h + b1_ref[0], 0.0)
        y = jnp.dot(h, w2_ref[0], preferred_element_type=jnp.float32)
        y = (y + b2_ref[0]) * ws_ref[:, 0:1]
        rows = jax.lax.broadcasted_iota(jnp.int32, (BLK, 1), 0)
        mask = (rows >= lo) & (rows < hi)
        ys_ref[...] += jnp.where(mask, y, 0.0)


def _gmm(xs, ws, W1, b1, W2, b2, tiles, groups, los, his):
    grid_spec = pltpu.PrefetchScalarGridSpec(
        num_scalar_prefetch=4,
        grid=(NS,),
        in_specs=[
            pl.BlockSpec((BLK, DIM), lambda s, t, g, l, h: (t[s], 0)),
            pl.BlockSpec((BLK, 128), lambda s, t, g, l, h: (t[s], 0)),
            pl.BlockSpec((1, DIM, HID), lambda s, t, g, l, h: (g[s], 0, 0)),
            pl.BlockSpec((1, 1, HID), lambda s, t, g, l, h: (g[s], 0, 0)),
            pl.BlockSpec((1, HID, DIM), lambda s, t, g, l, h: (g[s], 0, 0)),
            pl.BlockSpec((1, 1, DIM), lambda s, t, g, l, h: (g[s], 0, 0)),
        ],
        out_specs=pl.BlockSpec((BLK, DIM), lambda s, t, g, l, h: (t[s], 0)),
    )
    return pl.pallas_call(
        _gmm_body,
        grid_spec=grid_spec,
        out_shape=jax.ShapeDtypeStruct((NA, DIM), jnp.float32),
        compiler_params=pltpu.CompilerParams(vmem_limit_bytes=100 * 2**20),
    )(tiles, groups, los, his, xs, ws, W1, b1.reshape(NE, 1, HID), W2,
      b2.reshape(NE, 1, DIM))


def kernel(x, W1, b1, W2, b2, Wg, bg):
    orig_shape = x.shape
    xf = x.reshape(-1, orig_shape[-1])
    dispatch, combine = _sc_kernels()
    loss, position, w01, meta = _gate(xf, Wg, bg)
    pos2d = position.reshape(NA // 32, 32)
    xs, ws = dispatch(xf, w01, pos2d)
    ys = _gmm(xs, ws, W1, b1, W2, b2, meta[0], meta[1], meta[2], meta[3])
    out = combine(ys, pos2d)
    return out.reshape(orig_shape), loss.reshape(())


# speedup vs baseline: 1.9689x; 1.0083x over previous
import functools

import jax
import jax.numpy as jnp
from jax import lax
from jax.experimental import pallas as pl
from jax.experimental.pallas import tpu as pltpu
from jax.experimental.pallas import tpu_sc as plsc

DIM = 1024
HID = 2048
NE = 8
NTOK = 2048
NA = 2 * NTOK
BLK = 256
NT = NA // BLK
NS = NA // BLK + NE - 1

NSC_CORES = 2
NSC_SUB = 16
NW = NSC_CORES * NSC_SUB


def _tr8(row):
    eye = (jax.lax.broadcasted_iota(jnp.int32, (NE, NE), 0)
           == jax.lax.broadcasted_iota(jnp.int32, (NE, NE), 1)).astype(jnp.float32)
    return jnp.sum(jnp.broadcast_to(row, (NE, NE)) * eye, axis=1, keepdims=True)


def _gate_body(x_ref, wg_ref, bg_ref, loss_ref, pos_ref, w01_ref, meta_ref):
    x = x_ref[...]
    logits = jnp.dot(x, wg_ref[...], preferred_element_type=jnp.float32)
    logits = logits + bg_ref[...]
    m = jnp.max(logits, axis=-1, keepdims=True)
    ex = jnp.exp(logits - m)
    gw = ex / jnp.sum(ex, axis=-1, keepdims=True)

    lane = jax.lax.broadcasted_iota(jnp.int32, (NTOK, NE), 1)
    m1 = jnp.max(gw, axis=-1, keepdims=True)
    i1 = jnp.min(jnp.where(gw == m1, lane, NE), axis=-1, keepdims=True)
    sel1 = lane == i1
    gw2 = jnp.where(sel1, -jnp.inf, gw)
    m2 = jnp.max(gw2, axis=-1, keepdims=True)
    i2 = jnp.min(jnp.where(gw2 == m2, lane, NE), axis=-1, keepdims=True)
    sel2 = lane == i2

    wsum = m1 + m2
    w01_ref[...] = jnp.broadcast_to(
        jnp.concatenate([m1 / wsum, m2 / wsum], axis=0), (NA, 128))

    usage = jnp.sum(jnp.where(sel1 | sel2, 1.0, 0.0), axis=0, keepdims=True)
    prob = jnp.sum(gw, axis=0, keepdims=True) / NTOK
    loss_ref[...] = jnp.sum(prob * usage / NTOK).reshape(1, 1)

    oh = jnp.concatenate([sel1, sel2], axis=0).astype(jnp.float32)
    CH = 512
    rc = jax.lax.broadcasted_iota(jnp.int32, (CH, CH), 0)
    cc = jax.lax.broadcasted_iota(jnp.int32, (CH, CH), 1)
    ltc = (cc < rc).astype(jnp.float32)
    carry = jnp.zeros((1, NE), jnp.float32)
    rank_chunks = []
    for k in range(NA // CH):
        ohk = oh[k * CH:(k + 1) * CH]
        rk = jnp.dot(ltc, ohk, preferred_element_type=jnp.float32) + carry
        rank_chunks.append(jnp.sum(ohk * rk, axis=1, keepdims=True))
        carry = carry + jnp.sum(ohk, axis=0, keepdims=True)
    rank = jnp.concatenate(rank_chunks, axis=0)
    counts = carry

    r8 = jax.lax.broadcasted_iota(jnp.int32, (NE, NE), 0)
    c8 = jax.lax.broadcasted_iota(jnp.int32, (NE, NE), 1)
    strict_lt = (r8 < c8).astype(jnp.float32)

    def _excl_prefix_row(row):
        col = _tr8(row)
        return jnp.sum(jnp.broadcast_to(col, (NE, NE)) * strict_lt,
                       axis=0, keepdims=True)

    offs = _excl_prefix_row(counts)

    posf = rank + jnp.sum(oh * offs, axis=1, keepdims=True)
    pos_ref[...] = posf.astype(jnp.int32)

    t_start = jnp.floor(offs / BLK)
    endf = offs + counts
    t_end = jnp.floor((endf + (BLK - 1)) / BLK)
    nt = jnp.where(counts > 0, t_end - t_start, 0.0)
    so = _excl_prefix_row(nt)
    ce = so + nt
    total = jnp.sum(nt)

    ce_col = _tr8(ce)
    so_col = _tr8(so)
    ts_col = _tr8(t_start)
    off_col = _tr8(offs)
    end_col = _tr8(endf)

    siota = jax.lax.broadcasted_iota(jnp.int32, (1, NS), 1).astype(jnp.float32)
    pad = siota >= total
    s_c = jnp.minimum(siota, jnp.maximum(total - 1.0, 0.0))
    g = jnp.sum((s_c >= ce_col).astype(jnp.float32), axis=0, keepdims=True)

    e_col = jax.lax.broadcasted_iota(jnp.int32, (NE, 1), 0).astype(jnp.float32)
    gsel = (e_col == g).astype(jnp.float32)
    so_g = jnp.sum(gsel * so_col, axis=0, keepdims=True)
    ts_g = jnp.sum(gsel * ts_col, axis=0, keepdims=True)
    off_g = jnp.sum(gsel * off_col, axis=0, keepdims=True)
    end_g = jnp.sum(gsel * end_col, axis=0, keepdims=True)

    tile = s_c - so_g + ts_g
    lo = jnp.maximum(off_g, tile * BLK)
    hi = jnp.minimum(end_g, (tile + 1.0) * BLK)
    hi = jnp.where(pad, lo, hi)

    meta_ref[...] = jnp.concatenate([tile, g, lo, hi], axis=0).astype(jnp.int32)


def _gate(xf, Wg, bg):
    return pl.pallas_call(
        _gate_body,
        out_shape=(
            jax.ShapeDtypeStruct((1, 1), jnp.float32),
            jax.ShapeDtypeStruct((NA, 1), jnp.int32),
            jax.ShapeDtypeStruct((NA, 128), jnp.float32),
            jax.ShapeDtypeStruct((4, NS), jnp.int32),
        ),
    )(xf, Wg, bg.reshape(1, NE))


@functools.cache
def _sc_kernels():
    mesh = plsc.VectorSubcoreMesh(core_axis_name="c", subcore_axis_name="s")

    @functools.partial(
        pl.kernel,
        out_type=(
            jax.ShapeDtypeStruct((NA, DIM), jnp.float32),
            jax.ShapeDtypeStruct((NA, 128), jnp.float32),
        ),
        mesh=mesh,
        scratch_types=[
            pltpu.VMEM((4, 32), jnp.int32),
            pltpu.VMEM((32, DIM), jnp.float32),
            pltpu.VMEM((32, 128), jnp.float32),
            pltpu.SemaphoreType.DMA,
        ],
    )
    def _dispatch(x_hbm, w_hbm, pos_hbm, xs_hbm, ws_hbm, idx_v, buf, wbuf,
                  sem):
        wid = lax.axis_index("s") * NSC_CORES + lax.axis_index("c")
        pltpu.sync_copy(pos_hbm.at[pl.ds(wid * 4, 4)], idx_v)
        for c in range(4):
            j0 = wid * 128 + c * 32
            tok = lax.rem(j0, NTOK)
            pltpu.sync_copy(x_hbm.at[pl.ds(tok, 32)], buf)
            pltpu.sync_copy(w_hbm.at[pl.ds(j0, 32)], wbuf)
            h0 = pltpu.async_copy(buf, xs_hbm.at[idx_v.at[c]], sem)
            h1 = pltpu.async_copy(wbuf, ws_hbm.at[idx_v.at[c]], sem)
            h0.wait()
            h1.wait()

    @functools.partial(
        pl.kernel,
        out_type=jax.ShapeDtypeStruct((NTOK, DIM), jnp.float32),
        mesh=mesh,
        scratch_types=[
            pltpu.VMEM((2, 32), jnp.int32),
            pltpu.VMEM((32, DIM), jnp.float32),
            pltpu.VMEM((32, DIM), jnp.float32),
            pltpu.SemaphoreType.DMA,
        ],
    )
    def _combine(ys_hbm, pos_hbm, out_hbm, idx_v, buf0, buf1, sem):
        wid = lax.axis_index("s") * NSC_CORES + lax.axis_index("c")
        for c in range(2):
            n0 = wid * 64 + c * 32
            r0 = 2 * wid + c
            r1 = 64 + r0
            pltpu.sync_copy(pos_hbm.at[pl.ds(r0, 1)], idx_v.at[pl.ds(0, 1)])
            pltpu.sync_copy(pos_hbm.at[pl.ds(r1, 1)], idx_v.at[pl.ds(1, 1)])
            h0 = pltpu.async_copy(ys_hbm.at[idx_v.at[0]], buf0, sem)
            h1 = pltpu.async_copy(ys_hbm.at[idx_v.at[1]], buf1, sem)
            h0.wait()
            h1.wait()

            def row_body(r, carry):
                for t in range(DIM // 16):
                    a = buf0[r, pl.ds(t * 16, 16)]
                    b = buf1[r, pl.ds(t * 16, 16)]
                    buf0[r, pl.ds(t * 16, 16)] = a + b
                return carry

            lax.fori_loop(0, 32, row_body, 0)
            pltpu.sync_copy(buf0, out_hbm.at[pl.ds(n0, 32)])

    return _dispatch, _combine


def _gmm_body(tiles, groups, los, his, xs_ref, ws_ref, w1_ref, b1_ref, w2_ref,
              b2_ref, ys_ref):
    s = pl.program_id(0)
    tile = tiles[s]
    prev = tiles[jnp.maximum(s - 1, 0)]
    first = jnp.logical_or(s == 0, tile != prev)
    lo = los[s] - tile * BLK
    hi = his[s] - tile * BLK

    @pl.when(first)
    def _():
        ys_ref[...] = jnp.zeros_like(ys_ref)

    @pl.when(lo < hi)
    def _():
        h = jnp.dot(xs_ref[...], w1_ref[0], preferred_element_type=jnp.float32)
        h = jnp.maximum(h + b1_ref[0], 0.0)
        y = jnp.dot(h, w2_ref[0], preferred_element_type=jnp.float32)
        y = (y + b2_ref[0]) * ws_ref[:, 0:1]
        rows = jax.lax.broadcasted_iota(jnp.int32, (BLK, 1), 0)
        mask = (rows >= lo) & (rows < hi)
        ys_ref[...] += jnp.where(mask, y, 0.0)


def _gmm(xs, ws, W1, b1, W2, b2, tiles, groups, los, his):
    grid_spec = pltpu.PrefetchScalarGridSpec(
        num_scalar_prefetch=4,
        grid=(NS,),
        in_specs=[
            pl.BlockSpec((BLK, DIM), lambda s, t, g, l, h: (t[s], 0)),
            pl.BlockSpec((BLK, 128), lambda s, t, g, l, h: (t[s], 0)),
            pl.BlockSpec((1, DIM, HID), lambda s, t, g, l, h: (g[s], 0, 0)),
            pl.BlockSpec((1, 1, HID), lambda s, t, g, l, h: (g[s], 0, 0)),
            pl.BlockSpec((1, HID, DIM), lambda s, t, g, l, h: (g[s], 0, 0)),
            pl.BlockSpec((1, 1, DIM), lambda s, t, g, l, h: (g[s], 0, 0)),
        ],
        out_specs=pl.BlockSpec((BLK, DIM), lambda s, t, g, l, h: (t[s], 0)),
    )
    return pl.pallas_call(
        _gmm_body,
        grid_spec=grid_spec,
        out_shape=jax.ShapeDtypeStruct((NA, DIM), jnp.float32),
        compiler_params=pltpu.CompilerParams(vmem_limit_bytes=100 * 2**20),
    )(tiles, groups, los, his, xs, ws, W1, b1.reshape(NE, 1, HID), W2,
      b2.reshape(NE, 1, DIM))


def kernel(x, W1, b1, W2, b2, Wg, bg):
    orig_shape = x.shape
    xf = x.reshape(-1, orig_shape[-1])
    dispatch, combine = _sc_kernels()
    loss, position, w01, meta = _gate(xf, Wg, bg)
    pos2d = position.reshape(NA // 32, 32)
    xs, ws = dispatch(xf, w01, pos2d)
    ys = _gmm(xs, ws, W1, b1, W2, b2, meta[0], meta[1], meta[2], meta[3])
    out = combine(ys, pos2d)
    return out.reshape(orig_shape), loss.reshape(())
